# baseline (speedup 1.0000x reference)
import functools

import jax
import jax.numpy as jnp
import numpy as np
from jax import lax
from jax.experimental import pallas as pl
from jax.experimental.pallas import tpu as pltpu
from jax.experimental.pallas import tpu_sc as plsc

_B, _N, _T = 64, 32, 128
_S = _T * _N
_NW = 32
_CH = _S // _NW
_FL = _CH * _B
_SOCIAL_DIST = 1.5
_COHESION = 0.8
_NC = 2


_U32 = np.uint32


def _rotl(x, d):
    return ((x << _U32(d)) | (x >> _U32(32 - d))).astype(_U32)


def _threefry2x32(ks, count):
    flat = count.ravel().astype(_U32)
    odd = flat.size % 2
    if odd:
        flat = np.concatenate([flat, np.zeros(1, _U32)])
    x0, x1 = (h.copy() for h in np.split(flat, 2))
    k0, k1 = _U32(ks[0]), _U32(ks[1])
    k2 = _U32(k0 ^ k1 ^ _U32(0x1BD11BDA))
    kk = [k0, k1, k2]
    rotations = [(13, 15, 26, 6), (17, 29, 16, 24)]
    x0 = (x0 + k0).astype(_U32)
    x1 = (x1 + k1).astype(_U32)
    for i in range(5):
        for r in rotations[i % 2]:
            x0 = (x0 + x1).astype(_U32)
            x1 = _rotl(x1, r)
            x1 = (x1 ^ x0).astype(_U32)
        x0 = (x0 + kk[(i + 1) % 3]).astype(_U32)
        x1 = (x1 + kk[(i + 2) % 3] + _U32(i + 1)).astype(_U32)
    out = np.concatenate([x0, x1])
    if odd:
        out = out[:-1]
    return out.reshape(count.shape)


def _tf_pairs(ks, x0, x1):
    n = x0.size
    flat = np.concatenate([x0.ravel(), x1.ravel()]).astype(_U32)
    out = _threefry2x32(ks, flat)
    return out[:n], out[n:]


def _np_split(ks, num=2):
    o0, o1 = _tf_pairs(ks, np.zeros(num, _U32), np.arange(num, dtype=_U32))
    return np.stack([o0, o1], axis=1)


def _np_random_bits(ks, shape):
    size = int(np.prod(shape))
    o0, o1 = _tf_pairs(ks, np.zeros(size, _U32), np.arange(size, dtype=_U32))
    return (o0 ^ o1).reshape(shape)


def _np_randint(ks, shape, minval, maxval):
    k1, k2 = _np_split(ks)
    higher = _np_random_bits(k1, shape)
    lower = _np_random_bits(k2, shape)
    span = _U32(maxval - minval)
    multiplier = _U32((65536 % int(span)) ** 2 % int(span))
    result = ((higher % span) * multiplier + lower % span) % span
    return (minval + result.astype(np.int32)).astype(np.int32)


def _np_uniform01(ks, shape):
    bits = _np_random_bits(ks, shape)
    f = ((bits >> _U32(9)) | _U32(0x3F800000)).view(np.float32) - np.float32(1.0)
    return np.maximum(np.float32(0.0), f)


@functools.lru_cache(maxsize=1)
def _index_lists():
    rk1, rk2 = _np_split(np.array([0, 1234], _U32))
    rand_sel = _np_randint(rk1, (_S, _B, 1), 0, _B - 1)[:, :, 0]
    drop = _np_uniform01(rk2, (_S, _B)) < _COHESION
    b = np.arange(_B)[None, :]
    rand_nb = np.where(rand_sel < b, rand_sel, rand_sel + 1).astype(np.int32)

    drop_w = drop.reshape(_NW, _CH, _B)
    rand_w = rand_nb.reshape(_NW, _CH, _B)

    n_a = int(drop_w.reshape(_NW, -1).sum(axis=1).max())
    n_b = int((~drop_w).reshape(_NW, -1).sum(axis=1).max())
    n_a = -(-n_a // 64) * 64
    n_b = -(-n_b // 16) * 16

    a_self = np.zeros((_NW, n_a), np.int32)
    a_src = np.zeros((_NW, n_a), np.int32)
    a_dst = np.zeros((_NW, n_a), np.int32)
    b_base = np.zeros((_NW, n_b), np.int32)
    b_agent = np.zeros((_NW, n_b), np.int32)
    b_wr = np.zeros((_NW, n_b), np.int32)
    lane16 = np.arange(16, dtype=np.int32)

    for w in range(_NW):
        ss, bb = np.nonzero(drop_w[w])
        k = len(ss)
        a_self[w, :k] = ss * _B + bb
        a_src[w, :k] = ss * _B + rand_w[w, ss, bb]
        a_dst[w, :k] = ss * _B + bb
        a_dst[w, k:] = _FL + np.resize(lane16, n_a - k)

        ss, bb = np.nonzero(~drop_w[w])
        k = len(ss)
        b_base[w, :k] = ss * _B
        b_agent[w, :k] = bb
        b_wr[w, :k] = ss * _B + bb
        b_wr[w, k:] = _FL + np.resize(lane16, n_b - k)

    return a_self, a_src, a_dst, b_base, b_agent, b_wr


def _sc_body(px_hbm, py_hbm, coef_hbm, aself_hbm, asrc_hbm, adst_hbm,
             bbase_hbm, bagent_hbm, bwr_hbm, out_hbm,
             px_v, py_v, pxw_v, pyw_v, out_v, coef_v,
             aself_v, asrc_v, adst_v, bbase_v, bagent_v, bwr_v, dma_sem):
    wid = lax.axis_index("s") * _NC + lax.axis_index("c")

    copies = [
        pltpu.async_copy(px_hbm.at[wid], px_v, dma_sem),
        pltpu.async_copy(py_hbm.at[wid], py_v, dma_sem),
        pltpu.async_copy(coef_hbm, coef_v, dma_sem),
        pltpu.async_copy(aself_hbm.at[wid], aself_v, dma_sem),
        pltpu.async_copy(asrc_hbm.at[wid], asrc_v, dma_sem),
        pltpu.async_copy(adst_hbm.at[wid], adst_v, dma_sem),
        pltpu.async_copy(bbase_hbm.at[wid], bbase_v, dma_sem),
        pltpu.async_copy(bagent_hbm.at[wid], bagent_v, dma_sem),
        pltpu.async_copy(bwr_hbm.at[wid], bwr_v, dma_sem),
    ]
    for c in copies:
        c.wait()

    coefs = [[coef_v[pl.ds(kk * _B + k * 16, 16)] for k in range(4)]
             for kk in range(6)]

    def t_body(s, carry):
        base = s * _B
        for k in range(4):
            px = px_v[pl.ds(base + k * 16, 16)]
            py = py_v[pl.ds(base + k * 16, 16)]
            pxw_v[pl.ds(base + k * 16, 16)] = (
                coefs[0][k] * px + coefs[1][k] * py + coefs[2][k])
            pyw_v[pl.ds(base + k * 16, 16)] = (
                coefs[3][k] * px + coefs[4][k] * py + coefs[5][k])
        return carry

    lax.fori_loop(0, _CH, t_body, 0)

    n_a = aself_v.shape[0]

    def a_body(g, carry):
        for u in range(4):
            sl = pl.ds(g * 64 + u * 16, 16)
            self_i = aself_v[sl]
            src_i = asrc_v[sl]
            dst_i = adst_v[sl]
            xs = plsc.load_gather(pxw_v, [self_i])
            ys = plsc.load_gather(pyw_v, [self_i])
            xr = plsc.load_gather(pxw_v, [src_i])
            yr = plsc.load_gather(pyw_v, [src_i])
            dx = xs - xr
            dy = ys - yr
            plsc.store_scatter(out_v, [dst_i], dx * dx + dy * dy)
        return carry

    lax.fori_loop(0, n_a // 64, a_body, 0)

    n_b = bbase_v.shape[0]
    inf = jnp.float32(jnp.inf)

    def b_body(g, carry):
        sl = pl.ds(g * 16, 16)
        base = bbase_v[sl]
        agent = bagent_v[sl]
        wr = bwr_v[sl]
        self_i = base + agent
        xs = plsc.load_gather(pxw_v, [self_i])
        ys = plsc.load_gather(pyw_v, [self_i])

        accs = [jnp.full((16,), inf, jnp.float32) for _ in range(4)]
        for j in range(_B):
            idx = base + j
            xj = plsc.load_gather(pxw_v, [idx])
            yj = plsc.load_gather(pyw_v, [idx])
            dx = xs - xj
            dy = ys - yj
            d2 = dx * dx + dy * dy
            d2 = jnp.where(agent == j, inf, d2)
            accs[j % 4] = jnp.minimum(accs[j % 4], d2)
        acc = jnp.minimum(jnp.minimum(accs[0], accs[1]),
                          jnp.minimum(accs[2], accs[3]))
        plsc.store_scatter(out_v, [wr], acc)
        return carry

    lax.fori_loop(0, n_b // 16, b_body, 0)

    pltpu.sync_copy(out_v.at[pl.ds(0, _FL)], out_hbm.at[wid])


def _loss_body(nd2_ref, out_ref):
    nd = jnp.sqrt(nd2_ref[...])
    c = (nd - _SOCIAL_DIST) ** 2
    c3 = c.reshape(_T, _N, _B)
    out_ref[...] = jnp.sum(c3, axis=0) * (1.0 / _T)


def kernel(x, world_from_agent):
    a_self, a_src, a_dst, b_base, b_agent, b_wr = _index_lists()

    px = jnp.swapaxes(x[..., 0], 0, 2).reshape(_S, _B).reshape(_NW, _FL)
    py = jnp.swapaxes(x[..., 1], 0, 2).reshape(_S, _B).reshape(_NW, _FL)
    wfa = world_from_agent
    coef = jnp.concatenate([wfa[:, 0, 0], wfa[:, 0, 1], wfa[:, 0, 2],
                            wfa[:, 1, 0], wfa[:, 1, 1], wfa[:, 1, 2]])

    n_a = a_self.shape[1]
    n_b = b_base.shape[1]
    mesh = plsc.VectorSubcoreMesh(core_axis_name="c", subcore_axis_name="s",
                                  num_cores=_NC, num_subcores=_NW // _NC)
    sc_fn = pl.kernel(
        _sc_body, mesh=mesh,
        compiler_params=pltpu.CompilerParams(needs_layout_passes=False),
        out_type=jax.ShapeDtypeStruct((_NW, _FL), jnp.float32),
        scratch_types=[
            pltpu.VMEM((_FL,), jnp.float32),
            pltpu.VMEM((_FL,), jnp.float32),
            pltpu.VMEM((_FL,), jnp.float32),
            pltpu.VMEM((_FL,), jnp.float32),
            pltpu.VMEM((_FL + 16,), jnp.float32),
            pltpu.VMEM((6 * _B,), jnp.float32),
            pltpu.VMEM((n_a,), jnp.int32),
            pltpu.VMEM((n_a,), jnp.int32),
            pltpu.VMEM((n_a,), jnp.int32),
            pltpu.VMEM((n_b,), jnp.int32),
            pltpu.VMEM((n_b,), jnp.int32),
            pltpu.VMEM((n_b,), jnp.int32),
            pltpu.SemaphoreType.DMA,
        ],
    )
    nd2 = sc_fn(px, py, coef,
                jnp.asarray(a_self), jnp.asarray(a_src), jnp.asarray(a_dst),
                jnp.asarray(b_base), jnp.asarray(b_agent), jnp.asarray(b_wr))

    loss_nb = pl.pallas_call(
        _loss_body,
        out_shape=jax.ShapeDtypeStruct((_N, _B), jnp.float32),
    )(nd2.reshape(_S, _B))
    return loss_nb.T


# speedup vs baseline: 1.4156x; 1.4156x over previous
import functools

import jax
import jax.numpy as jnp
import numpy as np
from jax import lax
from jax.experimental import pallas as pl
from jax.experimental.pallas import tpu as pltpu
from jax.experimental.pallas import tpu_sc as plsc

_B, _N, _T = 64, 32, 128
_S = _T * _N
_NW = 32
_CH = _S // _NW
_FL = _CH * _B
_SOCIAL_DIST = 1.5
_COHESION = 0.8
_NC = 2


_U32 = np.uint32


def _rotl(x, d):
    return ((x << _U32(d)) | (x >> _U32(32 - d))).astype(_U32)


def _threefry2x32(ks, count):
    flat = count.ravel().astype(_U32)
    odd = flat.size % 2
    if odd:
        flat = np.concatenate([flat, np.zeros(1, _U32)])
    x0, x1 = (h.copy() for h in np.split(flat, 2))
    k0, k1 = _U32(ks[0]), _U32(ks[1])
    k2 = _U32(k0 ^ k1 ^ _U32(0x1BD11BDA))
    kk = [k0, k1, k2]
    rotations = [(13, 15, 26, 6), (17, 29, 16, 24)]
    x0 = (x0 + k0).astype(_U32)
    x1 = (x1 + k1).astype(_U32)
    for i in range(5):
        for r in rotations[i % 2]:
            x0 = (x0 + x1).astype(_U32)
            x1 = _rotl(x1, r)
            x1 = (x1 ^ x0).astype(_U32)
        x0 = (x0 + kk[(i + 1) % 3]).astype(_U32)
        x1 = (x1 + kk[(i + 2) % 3] + _U32(i + 1)).astype(_U32)
    out = np.concatenate([x0, x1])
    if odd:
        out = out[:-1]
    return out.reshape(count.shape)


def _tf_pairs(ks, x0, x1):
    n = x0.size
    flat = np.concatenate([x0.ravel(), x1.ravel()]).astype(_U32)
    out = _threefry2x32(ks, flat)
    return out[:n], out[n:]


def _np_split(ks, num=2):
    o0, o1 = _tf_pairs(ks, np.zeros(num, _U32), np.arange(num, dtype=_U32))
    return np.stack([o0, o1], axis=1)


def _np_random_bits(ks, shape):
    size = int(np.prod(shape))
    o0, o1 = _tf_pairs(ks, np.zeros(size, _U32), np.arange(size, dtype=_U32))
    return (o0 ^ o1).reshape(shape)


def _np_randint(ks, shape, minval, maxval):
    k1, k2 = _np_split(ks)
    higher = _np_random_bits(k1, shape)
    lower = _np_random_bits(k2, shape)
    span = _U32(maxval - minval)
    multiplier = _U32((65536 % int(span)) ** 2 % int(span))
    result = ((higher % span) * multiplier + lower % span) % span
    return (minval + result.astype(np.int32)).astype(np.int32)


def _np_uniform01(ks, shape):
    bits = _np_random_bits(ks, shape)
    f = ((bits >> _U32(9)) | _U32(0x3F800000)).view(np.float32) - np.float32(1.0)
    return np.maximum(np.float32(0.0), f)


@functools.lru_cache(maxsize=1)
def _index_lists():
    rk1, rk2 = _np_split(np.array([0, 1234], _U32))
    rand_sel = _np_randint(rk1, (_S, _B, 1), 0, _B - 1)[:, :, 0]
    drop = _np_uniform01(rk2, (_S, _B)) < _COHESION
    b = np.arange(_B)[None, :]
    rand_nb = np.where(rand_sel < b, rand_sel, rand_sel + 1).astype(np.int32)

    drop_w = drop.reshape(_NW, _CH, _B)
    rand_w = rand_nb.reshape(_NW, _CH, _B)

    n_a = int(drop_w.reshape(_NW, -1).sum(axis=1).max())
    n_b = int((~drop_w).reshape(_NW, -1).sum(axis=1).max())
    n_a = -(-n_a // 64) * 64
    n_b = -(-n_b // 16) * 16

    a_self = np.zeros((_NW, n_a), np.int32)
    a_src = np.zeros((_NW, n_a), np.int32)
    a_dst = np.zeros((_NW, n_a), np.int32)
    b_base = np.zeros((_NW, n_b), np.int32)
    b_agent = np.zeros((_NW, n_b), np.int32)
    b_wr = np.zeros((_NW, n_b), np.int32)
    lane16 = np.arange(16, dtype=np.int32)

    for w in range(_NW):
        ss, bb = np.nonzero(drop_w[w])
        k = len(ss)
        a_self[w, :k] = ss * _B + bb
        a_src[w, :k] = ss * _B + rand_w[w, ss, bb]
        a_dst[w, :k] = ss * _B + bb
        a_dst[w, k:] = _FL + np.resize(lane16, n_a - k)

        ss, bb = np.nonzero(~drop_w[w])
        k = len(ss)
        b_base[w, :k] = ss * _B
        b_agent[w, :k] = bb
        b_wr[w, :k] = ss * _B + bb
        b_wr[w, k:] = _FL + np.resize(lane16, n_b - k)

    return a_self, a_src, a_dst, b_base, b_agent, b_wr


def _sc_body(px_hbm, py_hbm, coef_hbm, aself_hbm, asrc_hbm, adst_hbm,
             bbase_hbm, bagent_hbm, bwr_hbm, out_hbm,
             px_v, py_v, pxw_v, pyw_v, out_v, coef_v,
             aself_v, asrc_v, adst_v, bbase_v, bagent_v, bwr_v, dma_sem):
    wid = lax.axis_index("s") * _NC + lax.axis_index("c")

    copies = [
        pltpu.async_copy(px_hbm.at[wid], px_v, dma_sem),
        pltpu.async_copy(py_hbm.at[wid], py_v, dma_sem),
        pltpu.async_copy(coef_hbm, coef_v, dma_sem),
        pltpu.async_copy(aself_hbm.at[wid], aself_v, dma_sem),
        pltpu.async_copy(asrc_hbm.at[wid], asrc_v, dma_sem),
        pltpu.async_copy(adst_hbm.at[wid], adst_v, dma_sem),
        pltpu.async_copy(bbase_hbm.at[wid], bbase_v, dma_sem),
        pltpu.async_copy(bagent_hbm.at[wid], bagent_v, dma_sem),
        pltpu.async_copy(bwr_hbm.at[wid], bwr_v, dma_sem),
    ]
    for c in copies:
        c.wait()

    coefs = [[coef_v[pl.ds(kk * _B + k * 16, 16)] for k in range(4)]
             for kk in range(6)]

    def t_body(s, carry):
        base = s * _B
        for k in range(4):
            px = px_v[pl.ds(base + k * 16, 16)]
            py = py_v[pl.ds(base + k * 16, 16)]
            pxw_v[pl.ds(base + k * 16, 16)] = (
                coefs[0][k] * px + coefs[1][k] * py + coefs[2][k])
            pyw_v[pl.ds(base + k * 16, 16)] = (
                coefs[3][k] * px + coefs[4][k] * py + coefs[5][k])
        return carry

    lax.fori_loop(0, _CH, t_body, 0)

    n_a = aself_v.shape[0]

    def a_body(g, carry):
        for u in range(4):
            sl = pl.ds(g * 64 + u * 16, 16)
            self_i = aself_v[sl]
            src_i = asrc_v[sl]
            dst_i = adst_v[sl]
            xs = plsc.load_gather(pxw_v, [self_i])
            ys = plsc.load_gather(pyw_v, [self_i])
            xr = plsc.load_gather(pxw_v, [src_i])
            yr = plsc.load_gather(pyw_v, [src_i])
            dx = xs - xr
            dy = ys - yr
            plsc.store_scatter(out_v, [dst_i], dx * dx + dy * dy)
        return carry

    lax.fori_loop(0, n_a // 64, a_body, 0)

    n_b = bbase_v.shape[0]
    inf = jnp.float32(jnp.inf)

    def b_body(g, carry):
        sl = pl.ds(g * 16, 16)
        base = bbase_v[sl]
        agent = bagent_v[sl]
        wr = bwr_v[sl]
        self_i = base + agent
        xs = plsc.load_gather(pxw_v, [self_i])
        ys = plsc.load_gather(pyw_v, [self_i])

        def j_body(jo, accs):
            accs = list(accs)
            for u in range(8):
                j = jo * 8 + u
                idx = base + j
                xj = plsc.load_gather(pxw_v, [idx])
                yj = plsc.load_gather(pyw_v, [idx])
                dx = xs - xj
                dy = ys - yj
                d2 = dx * dx + dy * dy
                d2 = jnp.where(agent == j, inf, d2)
                accs[u % 4] = jnp.minimum(accs[u % 4], d2)
            return tuple(accs)

        accs = lax.fori_loop(
            0, _B // 8, j_body,
            tuple(jnp.full((16,), inf, jnp.float32) for _ in range(4)))
        acc = jnp.minimum(jnp.minimum(accs[0], accs[1]),
                          jnp.minimum(accs[2], accs[3]))
        plsc.store_scatter(out_v, [wr], acc)
        return carry

    lax.fori_loop(0, n_b // 16, b_body, 0)

    pltpu.sync_copy(out_v.at[pl.ds(0, _FL)], out_hbm.at[wid])


def _loss_body(nd2_ref, out_ref):
    nd = jnp.sqrt(nd2_ref[...])
    c = (nd - _SOCIAL_DIST) ** 2
    c3 = c.reshape(_T, _N, _B)
    out_ref[...] = jnp.sum(c3, axis=0) * (1.0 / _T)


def kernel(x, world_from_agent):
    a_self, a_src, a_dst, b_base, b_agent, b_wr = _index_lists()

    px = jnp.swapaxes(x[..., 0], 0, 2).reshape(_S, _B).reshape(_NW, _FL)
    py = jnp.swapaxes(x[..., 1], 0, 2).reshape(_S, _B).reshape(_NW, _FL)
    wfa = world_from_agent
    coef = jnp.concatenate([wfa[:, 0, 0], wfa[:, 0, 1], wfa[:, 0, 2],
                            wfa[:, 1, 0], wfa[:, 1, 1], wfa[:, 1, 2]])

    n_a = a_self.shape[1]
    n_b = b_base.shape[1]
    mesh = plsc.VectorSubcoreMesh(core_axis_name="c", subcore_axis_name="s",
                                  num_cores=_NC, num_subcores=_NW // _NC)
    sc_fn = pl.kernel(
        _sc_body, mesh=mesh,
        compiler_params=pltpu.CompilerParams(needs_layout_passes=False),
        out_type=jax.ShapeDtypeStruct((_NW, _FL), jnp.float32),
        scratch_types=[
            pltpu.VMEM((_FL,), jnp.float32),
            pltpu.VMEM((_FL,), jnp.float32),
            pltpu.VMEM((_FL,), jnp.float32),
            pltpu.VMEM((_FL,), jnp.float32),
            pltpu.VMEM((_FL + 16,), jnp.float32),
            pltpu.VMEM((6 * _B,), jnp.float32),
            pltpu.VMEM((n_a,), jnp.int32),
            pltpu.VMEM((n_a,), jnp.int32),
            pltpu.VMEM((n_a,), jnp.int32),
            pltpu.VMEM((n_b,), jnp.int32),
            pltpu.VMEM((n_b,), jnp.int32),
            pltpu.VMEM((n_b,), jnp.int32),
            pltpu.SemaphoreType.DMA,
        ],
    )
    nd2 = sc_fn(px, py, coef,
                jnp.asarray(a_self), jnp.asarray(a_src), jnp.asarray(a_dst),
                jnp.asarray(b_base), jnp.asarray(b_agent), jnp.asarray(b_wr))

    loss_nb = pl.pallas_call(
        _loss_body,
        out_shape=jax.ShapeDtypeStruct((_N, _B), jnp.float32),
    )(nd2.reshape(_S, _B))
    return loss_nb.T


# speedup vs baseline: 1.4596x; 1.0311x over previous
import functools

import jax
import jax.numpy as jnp
import numpy as np
from jax import lax
from jax.experimental import pallas as pl
from jax.experimental.pallas import tpu as pltpu
from jax.experimental.pallas import tpu_sc as plsc

_B, _N, _T = 64, 32, 128
_S = _T * _N
_NW = 32
_CH = _S // _NW
_FL = _CH * _B
_SOCIAL_DIST = 1.5
_COHESION = 0.8
_NC = 2
_PAD = 256


_U32 = np.uint32


def _rotl(x, d):
    return ((x << _U32(d)) | (x >> _U32(32 - d))).astype(_U32)


def _threefry2x32(ks, count):
    flat = count.ravel().astype(_U32)
    odd = flat.size % 2
    if odd:
        flat = np.concatenate([flat, np.zeros(1, _U32)])
    x0, x1 = (h.copy() for h in np.split(flat, 2))
    k0, k1 = _U32(ks[0]), _U32(ks[1])
    k2 = _U32(k0 ^ k1 ^ _U32(0x1BD11BDA))
    kk = [k0, k1, k2]
    rotations = [(13, 15, 26, 6), (17, 29, 16, 24)]
    x0 = (x0 + k0).astype(_U32)
    x1 = (x1 + k1).astype(_U32)
    for i in range(5):
        for r in rotations[i % 2]:
            x0 = (x0 + x1).astype(_U32)
            x1 = _rotl(x1, r)
            x1 = (x1 ^ x0).astype(_U32)
        x0 = (x0 + kk[(i + 1) % 3]).astype(_U32)
        x1 = (x1 + kk[(i + 2) % 3] + _U32(i + 1)).astype(_U32)
    out = np.concatenate([x0, x1])
    if odd:
        out = out[:-1]
    return out.reshape(count.shape)


def _tf_pairs(ks, x0, x1):
    n = x0.size
    flat = np.concatenate([x0.ravel(), x1.ravel()]).astype(_U32)
    out = _threefry2x32(ks, flat)
    return out[:n], out[n:]


def _np_split(ks, num=2):
    o0, o1 = _tf_pairs(ks, np.zeros(num, _U32), np.arange(num, dtype=_U32))
    return np.stack([o0, o1], axis=1)


def _np_random_bits(ks, shape):
    size = int(np.prod(shape))
    o0, o1 = _tf_pairs(ks, np.zeros(size, _U32), np.arange(size, dtype=_U32))
    return (o0 ^ o1).reshape(shape)


def _np_randint(ks, shape, minval, maxval):
    k1, k2 = _np_split(ks)
    higher = _np_random_bits(k1, shape)
    lower = _np_random_bits(k2, shape)
    span = _U32(maxval - minval)
    multiplier = _U32((65536 % int(span)) ** 2 % int(span))
    result = ((higher % span) * multiplier + lower % span) % span
    return (minval + result.astype(np.int32)).astype(np.int32)


def _np_uniform01(ks, shape):
    bits = _np_random_bits(ks, shape)
    f = ((bits >> _U32(9)) | _U32(0x3F800000)).view(np.float32) - np.float32(1.0)
    return np.maximum(np.float32(0.0), f)


@functools.lru_cache(maxsize=1)
def _index_lists():
    rk1, rk2 = _np_split(np.array([0, 1234], _U32))
    rand_sel = _np_randint(rk1, (_S, _B, 1), 0, _B - 1)[:, :, 0]
    drop = _np_uniform01(rk2, (_S, _B)) < _COHESION
    b = np.arange(_B)[None, :]
    rand_nb = np.where(rand_sel < b, rand_sel, rand_sel + 1).astype(np.int32)

    drop_w = drop.reshape(_NW, _CH, _B)
    rand_w = rand_nb.reshape(_NW, _CH, _B)

    n_a = int(drop_w.reshape(_NW, -1).sum(axis=1).max())
    n_b = int((~drop_w).reshape(_NW, -1).sum(axis=1).max())
    n_a = -(-n_a // 64) * 64
    n_b = -(-n_b // 16) * 16

    a_self = np.zeros((_NW, n_a), np.int32)
    a_src = np.zeros((_NW, n_a), np.int32)
    a_dst = np.zeros((_NW, n_a), np.int32)
    b_base = np.zeros((_NW, n_b), np.int32)
    b_agent = np.zeros((_NW, n_b), np.int32)
    b_wr = np.zeros((_NW, n_b), np.int32)

    for w in range(_NW):
        ss, bb = np.nonzero(drop_w[w])
        k = len(ss)
        assert n_a - k <= _PAD and n_b - (_FL - k) <= _PAD
        a_self[w, :k] = ss * _B + bb
        a_src[w, :k] = ss * _B + rand_w[w, ss, bb]
        a_dst[w, :k] = ss * _B + bb
        a_dst[w, k:] = _FL + np.arange(n_a - k, dtype=np.int32)

        ss, bb = np.nonzero(~drop_w[w])
        k = len(ss)
        b_base[w, :k] = ss * _B
        b_agent[w, :k] = bb
        b_wr[w, :k] = ss * _B + bb
        b_wr[w, k:] = _FL + np.arange(n_b - k, dtype=np.int32)

    return a_self, a_src, a_dst, b_base, b_agent, b_wr


def _sc_body(px_hbm, py_hbm, coef_hbm, aself_hbm, asrc_hbm, adst_hbm,
             bbase_hbm, bagent_hbm, bwr_hbm, out_hbm,
             px_v, py_v, pxw_v, pyw_v, out_v, coef_v,
             aself_v, asrc_v, adst_v, bbase_v, bagent_v, bwr_v, dma_sem):
    wid = lax.axis_index("s") * _NC + lax.axis_index("c")

    copies = [
        pltpu.async_copy(px_hbm.at[wid], px_v, dma_sem),
        pltpu.async_copy(py_hbm.at[wid], py_v, dma_sem),
        pltpu.async_copy(coef_hbm, coef_v, dma_sem),
        pltpu.async_copy(aself_hbm.at[wid], aself_v, dma_sem),
        pltpu.async_copy(asrc_hbm.at[wid], asrc_v, dma_sem),
        pltpu.async_copy(adst_hbm.at[wid], adst_v, dma_sem),
        pltpu.async_copy(bbase_hbm.at[wid], bbase_v, dma_sem),
        pltpu.async_copy(bagent_hbm.at[wid], bagent_v, dma_sem),
        pltpu.async_copy(bwr_hbm.at[wid], bwr_v, dma_sem),
    ]
    for c in copies:
        c.wait()

    coefs = [[coef_v[pl.ds(kk * _B + k * 16, 16)] for k in range(4)]
             for kk in range(6)]

    @plsc.parallel_loop(0, _CH, 1, unroll=2)
    def t_body(s):
        base = s * _B
        for k in range(4):
            px = px_v[pl.ds(base + k * 16, 16)]
            py = py_v[pl.ds(base + k * 16, 16)]
            pxw_v[pl.ds(base + k * 16, 16)] = (
                coefs[0][k] * px + coefs[1][k] * py + coefs[2][k])
            pyw_v[pl.ds(base + k * 16, 16)] = (
                coefs[3][k] * px + coefs[4][k] * py + coefs[5][k])

    n_a = aself_v.shape[0]

    @plsc.parallel_loop(0, n_a // 16, 1, unroll=4)
    def a_body(g):
        sl = pl.ds(g * 16, 16)
        self_i = aself_v[sl]
        src_i = asrc_v[sl]
        dst_i = adst_v[sl]
        xs = plsc.load_gather(pxw_v, [self_i])
        ys = plsc.load_gather(pyw_v, [self_i])
        xr = plsc.load_gather(pxw_v, [src_i])
        yr = plsc.load_gather(pyw_v, [src_i])
        dx = xs - xr
        dy = ys - yr
        plsc.store_scatter(out_v, [dst_i], dx * dx + dy * dy)

    n_b = bbase_v.shape[0]
    inf = jnp.float32(jnp.inf)

    @plsc.parallel_loop(0, n_b // 16, 1, unroll=2)
    def b_body(g):
        sl = pl.ds(g * 16, 16)
        base = bbase_v[sl]
        agent = bagent_v[sl]
        wr = bwr_v[sl]
        self_i = base + agent
        xs = plsc.load_gather(pxw_v, [self_i])
        ys = plsc.load_gather(pyw_v, [self_i])

        def j_body(jo, accs):
            accs = list(accs)
            for u in range(8):
                j = jo * 8 + u
                idx = base + j
                xj = plsc.load_gather(pxw_v, [idx])
                yj = plsc.load_gather(pyw_v, [idx])
                dx = xs - xj
                dy = ys - yj
                d2 = dx * dx + dy * dy
                d2 = jnp.where(agent == j, inf, d2)
                accs[u % 4] = jnp.minimum(accs[u % 4], d2)
            return tuple(accs)

        accs = lax.fori_loop(
            0, _B // 8, j_body,
            tuple(jnp.full((16,), inf, jnp.float32) for _ in range(4)))
        acc = jnp.minimum(jnp.minimum(accs[0], accs[1]),
                          jnp.minimum(accs[2], accs[3]))
        plsc.store_scatter(out_v, [wr], acc)

    pltpu.sync_copy(out_v.at[pl.ds(0, _FL)], out_hbm.at[wid])


def _loss_body(nd2_ref, out_ref):
    nd = jnp.sqrt(nd2_ref[...])
    c = (nd - _SOCIAL_DIST) ** 2
    c3 = c.reshape(_T, _N, _B)
    out_ref[...] = jnp.sum(c3, axis=0) * (1.0 / _T)


def kernel(x, world_from_agent):
    a_self, a_src, a_dst, b_base, b_agent, b_wr = _index_lists()

    px = jnp.swapaxes(x[..., 0], 0, 2).reshape(_S, _B).reshape(_NW, _FL)
    py = jnp.swapaxes(x[..., 1], 0, 2).reshape(_S, _B).reshape(_NW, _FL)
    wfa = world_from_agent
    coef = jnp.concatenate([wfa[:, 0, 0], wfa[:, 0, 1], wfa[:, 0, 2],
                            wfa[:, 1, 0], wfa[:, 1, 1], wfa[:, 1, 2]])

    n_a = a_self.shape[1]
    n_b = b_base.shape[1]
    mesh = plsc.VectorSubcoreMesh(core_axis_name="c", subcore_axis_name="s",
                                  num_cores=_NC, num_subcores=_NW // _NC)
    sc_fn = pl.kernel(
        _sc_body, mesh=mesh,
        compiler_params=pltpu.CompilerParams(needs_layout_passes=False),
        out_type=jax.ShapeDtypeStruct((_NW, _FL), jnp.float32),
        scratch_types=[
            pltpu.VMEM((_FL,), jnp.float32),
            pltpu.VMEM((_FL,), jnp.float32),
            pltpu.VMEM((_FL,), jnp.float32),
            pltpu.VMEM((_FL,), jnp.float32),
            pltpu.VMEM((_FL + _PAD,), jnp.float32),
            pltpu.VMEM((6 * _B,), jnp.float32),
            pltpu.VMEM((n_a,), jnp.int32),
            pltpu.VMEM((n_a,), jnp.int32),
            pltpu.VMEM((n_a,), jnp.int32),
            pltpu.VMEM((n_b,), jnp.int32),
            pltpu.VMEM((n_b,), jnp.int32),
            pltpu.VMEM((n_b,), jnp.int32),
            pltpu.SemaphoreType.DMA,
        ],
    )
    nd2 = sc_fn(px, py, coef,
                jnp.asarray(a_self), jnp.asarray(a_src), jnp.asarray(a_dst),
                jnp.asarray(b_base), jnp.asarray(b_agent), jnp.asarray(b_wr))

    loss_nb = pl.pallas_call(
        _loss_body,
        out_shape=jax.ShapeDtypeStruct((_N, _B), jnp.float32),
    )(nd2.reshape(_S, _B))
    return loss_nb.T


# speedup vs baseline: 2.9415x; 2.0153x over previous
import functools

import jax
import jax.numpy as jnp
import numpy as np
from jax import lax
from jax.experimental import pallas as pl
from jax.experimental.pallas import tpu as pltpu
from jax.experimental.pallas import tpu_sc as plsc

_B, _N, _T = 64, 32, 128
_S = _T * _N
_NW = 32
_CH = _S // _NW
_FL = _CH * _B
_SOCIAL_DIST = 1.5
_COHESION = 0.8
_NC = 2
_PAD = 4096


_U32 = np.uint32


def _rotl(x, d):
    return ((x << _U32(d)) | (x >> _U32(32 - d))).astype(_U32)


def _threefry2x32(ks, count):
    flat = count.ravel().astype(_U32)
    odd = flat.size % 2
    if odd:
        flat = np.concatenate([flat, np.zeros(1, _U32)])
    x0, x1 = (h.copy() for h in np.split(flat, 2))
    k0, k1 = _U32(ks[0]), _U32(ks[1])
    k2 = _U32(k0 ^ k1 ^ _U32(0x1BD11BDA))
    kk = [k0, k1, k2]
    rotations = [(13, 15, 26, 6), (17, 29, 16, 24)]
    x0 = (x0 + k0).astype(_U32)
    x1 = (x1 + k1).astype(_U32)
    for i in range(5):
        for r in rotations[i % 2]:
            x0 = (x0 + x1).astype(_U32)
            x1 = _rotl(x1, r)
            x1 = (x1 ^ x0).astype(_U32)
        x0 = (x0 + kk[(i + 1) % 3]).astype(_U32)
        x1 = (x1 + kk[(i + 2) % 3] + _U32(i + 1)).astype(_U32)
    out = np.concatenate([x0, x1])
    if odd:
        out = out[:-1]
    return out.reshape(count.shape)


def _tf_pairs(ks, x0, x1):
    n = x0.size
    flat = np.concatenate([x0.ravel(), x1.ravel()]).astype(_U32)
    out = _threefry2x32(ks, flat)
    return out[:n], out[n:]


def _np_split(ks, num=2):
    o0, o1 = _tf_pairs(ks, np.zeros(num, _U32), np.arange(num, dtype=_U32))
    return np.stack([o0, o1], axis=1)


def _np_random_bits(ks, shape):
    size = int(np.prod(shape))
    o0, o1 = _tf_pairs(ks, np.zeros(size, _U32), np.arange(size, dtype=_U32))
    return (o0 ^ o1).reshape(shape)


def _np_randint(ks, shape, minval, maxval):
    k1, k2 = _np_split(ks)
    higher = _np_random_bits(k1, shape)
    lower = _np_random_bits(k2, shape)
    span = _U32(maxval - minval)
    multiplier = _U32((65536 % int(span)) ** 2 % int(span))
    result = ((higher % span) * multiplier + lower % span) % span
    return (minval + result.astype(np.int32)).astype(np.int32)


def _np_uniform01(ks, shape):
    bits = _np_random_bits(ks, shape)
    f = ((bits >> _U32(9)) | _U32(0x3F800000)).view(np.float32) - np.float32(1.0)
    return np.maximum(np.float32(0.0), f)


@functools.lru_cache(maxsize=1)
def _index_lists():
    rk1, rk2 = _np_split(np.array([0, 1234], _U32))
    rand_sel = _np_randint(rk1, (_S, _B, 1), 0, _B - 1)[:, :, 0]
    drop = _np_uniform01(rk2, (_S, _B)) < _COHESION
    b = np.arange(_B)[None, :]
    rand_nb = np.where(rand_sel < b, rand_sel, rand_sel + 1).astype(np.int32)

    drop_w = drop.reshape(_NW, _CH, _B)
    rand_w = rand_nb.reshape(_NW, _CH, _B)

    n_a = int(drop_w.reshape(_NW, -1).sum(axis=1).max())
    n_a = -(-n_a // 64) * 64

    keep_w = ~drop_w

    a_self = np.zeros((_NW, n_a), np.int32)
    a_src = np.zeros((_NW, n_a), np.int32)
    a_dst = np.zeros((_NW, n_a), np.int32)
    g_agent = np.zeros((_NW, _CH * 2 * 16), np.int32)
    g_wr = np.zeros((_NW, _CH * 2 * 16), np.int32)
    g_cnt = np.zeros((_NW, _CH * 16), np.int32)

    for w in range(_NW):
        ss, bb = np.nonzero(drop_w[w])
        k = len(ss)
        assert n_a - k <= _PAD
        a_self[w, :k] = ss * _B + bb
        a_src[w, :k] = ss * _B + rand_w[w, ss, bb]
        a_dst[w, :k] = ss * _B + bb
        a_dst[w, k:] = _FL + np.arange(n_a - k, dtype=np.int32)

        pad = 0
        for s in range(_CH):
            agents = np.nonzero(keep_w[w, s])[0]
            assert 0 < len(agents) <= 32
            g_cnt[w, s * 16:(s + 1) * 16] = len(agents)
            for q in range(2):
                chunk = agents[q * 16:q * 16 + 16]
                base = (s * 2 + q) * 16
                g_agent[w, base:base + len(chunk)] = chunk
                g_wr[w, base:base + len(chunk)] = s * _B + chunk
                for t in range(len(chunk), 16):
                    g_wr[w, base + t] = _FL + pad
                    pad += 1
        assert pad <= _PAD, pad

    return a_self, a_src, a_dst, g_cnt, g_agent, g_wr


def _sc_body(px_hbm, py_hbm, coef_hbm, aself_hbm, asrc_hbm, adst_hbm,
             gcnt_hbm, gagent_hbm, gwr_hbm, out_hbm,
             px_v, py_v, pxw_v, pyw_v, out_v, coef_v,
             aself_v, asrc_v, adst_v, gcnt_v, gagent_v, gwr_v, dma_sem):
    wid = lax.axis_index("s") * _NC + lax.axis_index("c")

    copies = [
        pltpu.async_copy(px_hbm.at[wid], px_v, dma_sem),
        pltpu.async_copy(py_hbm.at[wid], py_v, dma_sem),
        pltpu.async_copy(coef_hbm, coef_v, dma_sem),
        pltpu.async_copy(aself_hbm.at[wid], aself_v, dma_sem),
        pltpu.async_copy(asrc_hbm.at[wid], asrc_v, dma_sem),
        pltpu.async_copy(adst_hbm.at[wid], adst_v, dma_sem),
        pltpu.async_copy(gcnt_hbm.at[wid], gcnt_v, dma_sem),
        pltpu.async_copy(gagent_hbm.at[wid], gagent_v, dma_sem),
        pltpu.async_copy(gwr_hbm.at[wid], gwr_v, dma_sem),
    ]
    for c in copies:
        c.wait()

    coefs = [[coef_v[pl.ds(kk * _B + k * 16, 16)] for k in range(4)]
             for kk in range(6)]

    @plsc.parallel_loop(0, _CH, 1, unroll=2)
    def t_body(s):
        base = s * _B
        for k in range(4):
            px = px_v[pl.ds(base + k * 16, 16)]
            py = py_v[pl.ds(base + k * 16, 16)]
            pxw_v[pl.ds(base + k * 16, 16)] = (
                coefs[0][k] * px + coefs[1][k] * py + coefs[2][k])
            pyw_v[pl.ds(base + k * 16, 16)] = (
                coefs[3][k] * px + coefs[4][k] * py + coefs[5][k])

    n_a = aself_v.shape[0]

    @plsc.parallel_loop(0, n_a // 16, 1, unroll=4)
    def a_body(g):
        sl = pl.ds(g * 16, 16)
        self_i = aself_v[sl]
        src_i = asrc_v[sl]
        dst_i = adst_v[sl]
        xs = plsc.load_gather(pxw_v, [self_i])
        ys = plsc.load_gather(pyw_v, [self_i])
        xr = plsc.load_gather(pxw_v, [src_i])
        yr = plsc.load_gather(pyw_v, [src_i])
        dx = xs - xr
        dy = ys - yr
        plsc.store_scatter(out_v, [dst_i], dx * dx + dy * dy)

    inf = jnp.float32(jnp.inf)

    @plsc.parallel_loop(0, _CH, 1, unroll=1)
    def b_body(s):
        sbase = s * _B
        rowx = [pxw_v[pl.ds(sbase + 16 * k, 16)] for k in range(4)]
        rowy = [pyw_v[pl.ds(sbase + 16 * k, 16)] for k in range(4)]

        def do_group(q):
            gb = (s * 2 + q) * 16
            agent = gagent_v[pl.ds(gb, 16)]
            wr = gwr_v[pl.ds(gb, 16)]
            xs = plsc.load_gather(pxw_v, [agent + sbase])
            ys = plsc.load_gather(pyw_v, [agent + sbase])
            accs = [jnp.full((16,), inf, jnp.float32) for _ in range(4)]
            for k in range(4):
                rx, ry = rowx[k], rowy[k]
                for t in range(16):
                    j = k * 16 + t
                    xj = jnp.full((16,), rx[t])
                    yj = jnp.full((16,), ry[t])
                    dx = xs - xj
                    dy = ys - yj
                    d2 = dx * dx + dy * dy
                    d2 = jnp.where(agent == j, inf, d2)
                    accs[j % 4] = jnp.minimum(accs[j % 4], d2)
            acc = jnp.minimum(jnp.minimum(accs[0], accs[1]),
                              jnp.minimum(accs[2], accs[3]))
            plsc.store_scatter(out_v, [wr], acc)

        do_group(0)
        cnt = jnp.max(gcnt_v[pl.ds(s * 16, 16)])

        @pl.when(cnt > 16)
        def _():
            do_group(1)

    pltpu.sync_copy(out_v.at[pl.ds(0, _FL)], out_hbm.at[wid])


def _loss_body(nd2_ref, out_ref):
    nd = jnp.sqrt(nd2_ref[...])
    c = (nd - _SOCIAL_DIST) ** 2
    c3 = c.reshape(_T, _N, _B)
    out_ref[...] = jnp.sum(c3, axis=0) * (1.0 / _T)


def kernel(x, world_from_agent):
    a_self, a_src, a_dst, g_cnt, g_agent, g_wr = _index_lists()

    px = jnp.swapaxes(x[..., 0], 0, 2).reshape(_S, _B).reshape(_NW, _FL)
    py = jnp.swapaxes(x[..., 1], 0, 2).reshape(_S, _B).reshape(_NW, _FL)
    wfa = world_from_agent
    coef = jnp.concatenate([wfa[:, 0, 0], wfa[:, 0, 1], wfa[:, 0, 2],
                            wfa[:, 1, 0], wfa[:, 1, 1], wfa[:, 1, 2]])

    n_a = a_self.shape[1]
    mesh = plsc.VectorSubcoreMesh(core_axis_name="c", subcore_axis_name="s",
                                  num_cores=_NC, num_subcores=_NW // _NC)
    sc_fn = pl.kernel(
        _sc_body, mesh=mesh,
        compiler_params=pltpu.CompilerParams(needs_layout_passes=False),
        out_type=jax.ShapeDtypeStruct((_NW, _FL), jnp.float32),
        scratch_types=[
            pltpu.VMEM((_FL,), jnp.float32),
            pltpu.VMEM((_FL,), jnp.float32),
            pltpu.VMEM((_FL,), jnp.float32),
            pltpu.VMEM((_FL,), jnp.float32),
            pltpu.VMEM((_FL + _PAD,), jnp.float32),
            pltpu.VMEM((6 * _B,), jnp.float32),
            pltpu.VMEM((n_a,), jnp.int32),
            pltpu.VMEM((n_a,), jnp.int32),
            pltpu.VMEM((n_a,), jnp.int32),
            pltpu.VMEM((_CH * 16,), jnp.int32),
            pltpu.VMEM((_CH * 2 * 16,), jnp.int32),
            pltpu.VMEM((_CH * 2 * 16,), jnp.int32),
            pltpu.SemaphoreType.DMA,
        ],
    )
    nd2 = sc_fn(px, py, coef,
                jnp.asarray(a_self), jnp.asarray(a_src), jnp.asarray(a_dst),
                jnp.asarray(g_cnt), jnp.asarray(g_agent), jnp.asarray(g_wr))

    loss_nb = pl.pallas_call(
        _loss_body,
        out_shape=jax.ShapeDtypeStruct((_N, _B), jnp.float32),
    )(nd2.reshape(_S, _B))
    return loss_nb.T
